# baseline (speedup 1.0000x reference)
import functools

import jax
import jax.numpy as jnp
from jax.experimental import pallas as pl
from jax.experimental.pallas import tpu as pltpu

_EPS = 1e-5


def _conv_image(xe_ref, w_ref, *, kh, kw, wo, hw, pad_rows):
    parts = []
    for dj in range(kw):
        acc = None
        for di in range(kh):
            s = pad_rows + (di - (kh // 2)) * wo + (dj - (kw // 2))
            p = jnp.dot(xe_ref[s:s + hw, :], w_ref[di * kw + dj],
                        preferred_element_type=jnp.float32)
            acc = p if acc is None else acc + p
        parts.append(acc)

    col = jax.lax.broadcasted_iota(jnp.int32, (hw, 1), 0) % wo
    acc = parts[kw // 2]
    for dj in range(kw):
        if dj == kw // 2:
            continue
        off = dj - (kw // 2)
        if off < 0:
            good = (col >= -off).astype(jnp.float32)
        else:
            good = (col < wo - off).astype(jnp.float32)
        acc = acc + parts[dj] * good
    return acc


def _fused_kernel(x_ref, w_ref, g_ref, b_ref, o_ref, xe_ref, y_ref, st_ref,
                  *, kh, kw, wo, hw, pad_rows, imgs, blk, p1_steps, count):
    j = pl.program_id(0)

    @pl.when(j == 0)
    def _init():
        xe_ref[...] = jnp.zeros_like(xe_ref)
        st_ref[...] = jnp.zeros_like(st_ref)

    @pl.when(j < p1_steps)
    def _conv_phase():
        for i in range(imgs):
            xt = jnp.transpose(x_ref[i], (1, 0))
            xe_ref[i, pad_rows:pad_rows + hw, :] = xt
            acc = _conv_image(xe_ref.at[i], w_ref, kh=kh, kw=kw, wo=wo,
                              hw=hw, pad_rows=pad_rows)
            s1 = jnp.sum(acc, axis=0, keepdims=True)
            s2 = jnp.sum(acc * acc, axis=0, keepdims=True)
            st_ref[...] += jnp.concatenate([s1, s2], axis=0)
            y_ref[j * imgs + i] = jnp.transpose(acc, (1, 0)).astype(y_ref.dtype)

    @pl.when(j >= p1_steps)
    def _bn_phase():
        tot = st_ref[...]
        mean = tot[0] / count
        var = jnp.maximum(tot[1] / count - mean * mean, 0.0)
        scale = g_ref[0] * jax.lax.rsqrt(var + _EPS)
        shift = b_ref[0] - mean * scale
        scol = scale.reshape(-1, 1)
        bcol = shift.reshape(-1, 1)
        k = j - p1_steps
        yblk = y_ref[pl.ds(k * blk, blk)].astype(jnp.float32)
        o_ref[...] = jnp.maximum(yblk * scol + bcol, 0.0)


def kernel(x_nchw, conv_w, conv_b, gamma, beta):
    del conv_b
    N, Cin, H, W = x_nchw.shape
    Cout, cin2, kh, kw = conv_w.shape
    assert cin2 == Cin
    Ho, Wo = H, W
    hw = H * W
    pad_rows = (W + kw // 2 + 7) // 8 * 8
    count = float(N * Ho * Wo)
    imgs = next(b for b in (4, 2, 1) if N % b == 0)
    blk = next(b for b in (8, 4, 2, 1) if N % b == 0)
    p1_steps = N // imgs
    p2_steps = N // blk

    xf = x_nchw.astype(jnp.float32).reshape(N, Cin, hw)
    w9 = jnp.transpose(conv_w, (2, 3, 1, 0)).reshape(kh * kw, Cin, Cout)
    w9 = w9.astype(jnp.float32)

    out_flat = pl.pallas_call(
        functools.partial(_fused_kernel, kh=kh, kw=kw, wo=Wo, hw=hw,
                          pad_rows=pad_rows, imgs=imgs, blk=blk,
                          p1_steps=p1_steps, count=count),
        grid=(p1_steps + p2_steps,),
        in_specs=[
            pl.BlockSpec((imgs, Cin, hw),
                         lambda j: (jnp.minimum(j, p1_steps - 1), 0, 0)),
            pl.BlockSpec((kh * kw, Cin, Cout), lambda j: (0, 0, 0)),
            pl.BlockSpec((1, Cout), lambda j: (0, 0)),
            pl.BlockSpec((1, Cout), lambda j: (0, 0)),
        ],
        out_specs=pl.BlockSpec(
            (blk, Cout, hw),
            lambda j: (jnp.maximum(j - p1_steps, 0), 0, 0)),
        out_shape=jax.ShapeDtypeStruct((N, Cout, hw), jnp.float32),
        scratch_shapes=[
            pltpu.VMEM((imgs, hw + 2 * pad_rows, Cin), jnp.float32),
            pltpu.VMEM((N, Cout, hw), jnp.bfloat16),
            pltpu.VMEM((2, Cout), jnp.float32),
        ],
        compiler_params=pltpu.CompilerParams(
            dimension_semantics=("arbitrary",),
            vmem_limit_bytes=100 * 1024 * 1024,
        ),
    )(xf, w9, gamma.astype(jnp.float32).reshape(1, Cout),
      beta.astype(jnp.float32).reshape(1, Cout))

    return out_flat.reshape(N, Cout, Ho, Wo)


# speedup vs baseline: 1.0241x; 1.0241x over previous
import functools

import jax
import jax.numpy as jnp
from jax.experimental import pallas as pl
from jax.experimental.pallas import tpu as pltpu

_EPS = 1e-5


def _conv_image(xe_ref, w_ref, *, kh, kw, wo, hw, pad_rows):
    parts = []
    for dj in range(kw):
        acc = None
        for di in range(kh):
            s = pad_rows + (di - (kh // 2)) * wo + (dj - (kw // 2))
            p = jnp.dot(xe_ref[s:s + hw, :], w_ref[di * kw + dj],
                        preferred_element_type=jnp.float32)
            acc = p if acc is None else acc + p
        parts.append(acc)

    col = jax.lax.broadcasted_iota(jnp.int32, (hw, 1), 0) % wo
    acc = parts[kw // 2]
    for dj in range(kw):
        if dj == kw // 2:
            continue
        off = dj - (kw // 2)
        if off < 0:
            good = (col >= -off).astype(jnp.float32)
        else:
            good = (col < wo - off).astype(jnp.float32)
        acc = acc + parts[dj] * good
    return acc


def _fused_kernel(x_ref, w_ref, g_ref, b_ref, o_ref, xe_ref, y_ref, st_ref,
                  *, kh, kw, wo, hw, pad_rows, imgs, blk, p1_steps, count):
    j = pl.program_id(0)

    @pl.when(j == 0)
    def _init():
        xe_ref[...] = jnp.zeros_like(xe_ref)
        st_ref[...] = jnp.zeros_like(st_ref)

    @pl.when(j < p1_steps)
    def _conv_phase():
        for i in range(imgs):
            xt = jnp.transpose(x_ref[i], (1, 0))
            xe_ref[i, pad_rows:pad_rows + hw, :] = xt
            acc = _conv_image(xe_ref.at[i], w_ref, kh=kh, kw=kw, wo=wo,
                              hw=hw, pad_rows=pad_rows)
            s1 = jnp.sum(acc, axis=0, keepdims=True)
            s2 = jnp.sum(acc * acc, axis=0, keepdims=True)
            st_ref[...] += jnp.concatenate([s1, s2], axis=0)
            y_ref[j * imgs + i] = jnp.transpose(acc, (1, 0))

    @pl.when(j >= p1_steps)
    def _bn_phase():
        tot = st_ref[...]
        mean = tot[0] / count
        var = jnp.maximum(tot[1] / count - mean * mean, 0.0)
        scale = g_ref[0] * jax.lax.rsqrt(var + _EPS)
        shift = b_ref[0] - mean * scale
        scol = scale.reshape(-1, 1)
        bcol = shift.reshape(-1, 1)
        k = j - p1_steps
        yblk = y_ref[pl.ds(k * blk, blk)]
        o_ref[...] = jnp.maximum(yblk * scol + bcol, 0.0)


def kernel(x_nchw, conv_w, conv_b, gamma, beta):
    del conv_b
    N, Cin, H, W = x_nchw.shape
    Cout, cin2, kh, kw = conv_w.shape
    assert cin2 == Cin
    Ho, Wo = H, W
    hw = H * W
    pad_rows = (W + kw // 2 + 7) // 8 * 8
    count = float(N * Ho * Wo)
    imgs = next(b for b in (4, 2, 1) if N % b == 0)
    blk = next(b for b in (8, 4, 2, 1) if N % b == 0)
    p1_steps = N // imgs
    p2_steps = N // blk

    xf = x_nchw.astype(jnp.float32).reshape(N, Cin, hw)
    w9 = jnp.transpose(conv_w, (2, 3, 1, 0)).reshape(kh * kw, Cin, Cout)
    w9 = w9.astype(jnp.float32)

    out_flat = pl.pallas_call(
        functools.partial(_fused_kernel, kh=kh, kw=kw, wo=Wo, hw=hw,
                          pad_rows=pad_rows, imgs=imgs, blk=blk,
                          p1_steps=p1_steps, count=count),
        grid=(p1_steps + p2_steps,),
        in_specs=[
            pl.BlockSpec((imgs, Cin, hw),
                         lambda j: (jnp.minimum(j, p1_steps - 1), 0, 0)),
            pl.BlockSpec((kh * kw, Cin, Cout), lambda j: (0, 0, 0)),
            pl.BlockSpec((1, Cout), lambda j: (0, 0)),
            pl.BlockSpec((1, Cout), lambda j: (0, 0)),
        ],
        out_specs=pl.BlockSpec(
            (blk, Cout, hw),
            lambda j: (jnp.maximum(j - p1_steps, 0), 0, 0)),
        out_shape=jax.ShapeDtypeStruct((N, Cout, hw), jnp.float32),
        scratch_shapes=[
            pltpu.VMEM((imgs, hw + 2 * pad_rows, Cin), jnp.float32),
            pltpu.VMEM((N, Cout, hw), jnp.float32),
            pltpu.VMEM((2, Cout), jnp.float32),
        ],
        compiler_params=pltpu.CompilerParams(
            dimension_semantics=("arbitrary",),
            vmem_limit_bytes=100 * 1024 * 1024,
        ),
    )(xf, w9, gamma.astype(jnp.float32).reshape(1, Cout),
      beta.astype(jnp.float32).reshape(1, Cout))

    return out_flat.reshape(N, Cout, Ho, Wo)
